# baseline (speedup 1.0000x reference)
import functools

import jax
import jax.numpy as jnp
from jax import lax
from jax.experimental import pallas as pl
from jax.experimental.pallas import tpu as pltpu
from jax.experimental.pallas import tpu_sc as plsc

NC = 2
NS = 16
NW = NC * NS
CH = 128


def _transpose_kernel(V, D):
    mesh = plsc.VectorSubcoreMesh(core_axis_name="c", subcore_axis_name="s")
    PAN = 4 * CH
    n_full = V // CH
    tail = V - n_full * CH
    n_pan = n_full // 4
    base_cnt = n_pan // NW
    extra = n_pan - base_cnt * NW
    NB = 2

    @functools.partial(
        pl.kernel,
        mesh=mesh,
        compiler_params=pltpu.CompilerParams(needs_layout_passes=False),
        out_type=jax.ShapeDtypeStruct((V // 2, 2 * D), jnp.float32),
        scratch_types=[
            pltpu.VMEM((NB, D, PAN + 1), jnp.float32),
            pltpu.VMEM((4, CH // 2, 2 * D), jnp.float32),
            pltpu.SemaphoreType.DMA,
            pltpu.SemaphoreType.DMA,
        ],
    )
    def k(tt_hbm, tail_hbm, out_hbm, ibuf, obuf, isem, osem):
        wid = lax.axis_index("s") * NC + lax.axis_index("c")
        cnt = jnp.where(wid < extra, base_cnt + 1, base_cnt)
        start = wid * base_cnt + jnp.minimum(wid, extra)
        lanes = lax.iota(jnp.int32, 16)
        qlanes = [q * 16 + lanes for q in range(D // 16)]

        def start_panel(p, slot):
            for tr in range(D // 8):
                pltpu.async_copy(
                    tt_hbm.at[pl.ds(tr * 8, 8), pl.ds(p * PAN, PAN)],
                    ibuf.at[slot, pl.ds(tr * 8, 8), pl.ds(0, PAN)],
                    isem,
                )

        def do_panel(p, slot):
            for tr in range(D // 8):
                pltpu.make_async_copy(
                    tt_hbm.at[pl.ds(tr * 8, 8), pl.ds(p * PAN, PAN)],
                    ibuf.at[slot, pl.ds(tr * 8, 8), pl.ds(0, PAN)],
                    isem,
                ).wait()
            slot16 = jnp.full((16,), slot, jnp.int32)
            for sub in range(4):

                def row(vv, _):
                    for h in range(2):
                        col = jnp.full(
                            (16,), sub * CH + 2 * vv + h, jnp.int32
                        )
                        for q in range(D // 16):
                            v = plsc.load_gather(
                                ibuf, [slot16, qlanes[q], col]
                            )
                            obuf[sub, vv, pl.ds(h * D + q * 16, 16)] = v
                    return 0

                lax.fori_loop(0, CH // 2, row, 0, unroll=4)
                pltpu.async_copy(
                    obuf.at[sub],
                    out_hbm.at[
                        pl.ds((p * 4 + sub) * (CH // 2), CH // 2)
                    ],
                    osem,
                )

        def drain_panel(p):
            for sub in range(4):
                pltpu.make_async_copy(
                    obuf.at[sub],
                    out_hbm.at[
                        pl.ds((p * 4 + sub) * (CH // 2), CH // 2)
                    ],
                    osem,
                ).wait()

        @pl.when(0 < cnt)
        def _():
            start_panel(start, 0)

        def body(j, _):
            for u in range(NB):
                jj = j * NB + u

                @pl.when(jj < cnt)
                def _(jj=jj, u=u):
                    @pl.when(jj + 1 < cnt)
                    def _():
                        start_panel(jj + 1 + start, (u + 1) % NB)

                    @pl.when(jj >= 1)
                    def _():
                        drain_panel(jj - 1 + start)

                    do_panel(jj + start, u)
            return 0

        lax.fori_loop(0, (base_cnt + 1 + NB - 1) // NB, body, 0)
        @pl.when(cnt >= 1)
        def _():
            drain_panel(start + cnt - 1)

        if tail:
            @pl.when(wid == NW - 1)
            def _():
                pltpu.sync_copy(tail_hbm, obuf.at[0, pl.ds(0, tail // 2)])
                pltpu.sync_copy(
                    obuf.at[0, pl.ds(0, tail // 2)],
                    out_hbm.at[pl.ds(n_full * (CH // 2), tail // 2)],
                )

    return k


def _gather_kernel(B, V, D, n_chunks):
    mesh = plsc.VectorSubcoreMesh(core_axis_name="c", subcore_axis_name="s")
    NBUF = 5
    NSPLIT = 8
    SUB = CH // NSPLIT

    @functools.partial(
        pl.kernel,
        mesh=mesh,
        compiler_params=pltpu.CompilerParams(
            needs_layout_passes=False, use_tc_tiling_on_sc=False
        ),
        out_type=jax.ShapeDtypeStruct((B, 2 * D), jnp.float32),
        scratch_types=[
            pltpu.VMEM((n_chunks, CH), jnp.int32),
            pltpu.VMEM((NBUF, CH, D), jnp.float32),
            pltpu.VMEM((2, CH, 2 * D), jnp.float32),
            pltpu.SemaphoreType.DMA,
            pltpu.SemaphoreType.DMA,
        ],
    )
    def k(idx_hbm, tab_hbm, out_hbm, idx_v, buf, outb, gsem, osem):
        wid = lax.axis_index("s") * NC + lax.axis_index("c")
        base = wid * (n_chunks * CH)
        pltpu.sync_copy(idx_hbm.at[wid], idx_v)

        def start_chunk(j, slot):
            for t in range(NSPLIT):
                pltpu.async_copy(
                    tab_hbm.at[idx_v.at[j, pl.ds(t * SUB, SUB)]],
                    buf.at[slot, pl.ds(t * SUB, SUB)],
                    gsem,
                )

        def finish_chunk(j, slot, oslot):
            for t in range(NSPLIT):
                pltpu.make_async_copy(
                    tab_hbm.at[idx_v.at[j, pl.ds(t * SUB, SUB)]],
                    buf.at[slot, pl.ds(t * SUB, SUB)],
                    gsem,
                ).wait()

            def scale_row(r, _):
                for c in range(D // 16):
                    outb[oslot, r, pl.ds(c * 16, 16)] = (
                        buf[slot, r, pl.ds(c * 16, 16)] * 8.0
                    )
                return 0

            lax.fori_loop(0, CH, scale_row, 0, unroll=4)
            pltpu.async_copy(
                outb.at[oslot],
                out_hbm.at[pl.ds(base + j * CH, CH)],
                osem,
            )

        def drain_out(j, oslot):
            pltpu.make_async_copy(
                outb.at[oslot],
                out_hbm.at[pl.ds(base + j * CH, CH)],
                osem,
            ).wait()

        for u in range(NBUF - 1):
            start_chunk(u, u)

        def body(j, _):
            for u in range(NBUF):
                jj = j * NBUF + u
                oslot = lax.rem(jj, 2)

                @pl.when(jj + NBUF - 1 < n_chunks)
                def _(jj=jj, u=u):
                    start_chunk(jj + NBUF - 1, (u + NBUF - 1) % NBUF)

                @pl.when(jj >= 2)
                def _(jj=jj, oslot=oslot):
                    drain_out(jj - 2, oslot)

                finish_chunk(jj, u, oslot)
            return 0

        lax.fori_loop(0, n_chunks // NBUF, body, 0)
        drain_out(n_chunks - 2, lax.rem(n_chunks - 2, 2))
        drain_out(n_chunks - 1, lax.rem(n_chunks - 1, 2))

    return k


def kernel(x, table):
    S0, S1 = x.shape
    V, D = table.shape
    B = S0 * S1
    n_chunks = B // (NW * CH)
    xi = x.astype(jnp.int32)
    idx = xi.reshape(NW, n_chunks, CH)
    g = _gather_kernel(B, V, D, n_chunks)(idx, table)
    return g[:, :D].reshape(S0, S1, D)


# speedup vs baseline: 1.0036x; 1.0036x over previous
import functools

import jax
import jax.numpy as jnp
from jax import lax
from jax.experimental import pallas as pl
from jax.experimental.pallas import tpu as pltpu
from jax.experimental.pallas import tpu_sc as plsc

NC = 2
NS = 16
NW = NC * NS
CH = 128


def _gather_kernel(B, V, D, n_chunks):
    mesh = plsc.VectorSubcoreMesh(core_axis_name="c", subcore_axis_name="s")
    NBUF = 4
    NSPLIT = 4
    SUB = CH // NSPLIT

    @functools.partial(
        pl.kernel,
        mesh=mesh,
        compiler_params=pltpu.CompilerParams(
            needs_layout_passes=False, use_tc_tiling_on_sc=False
        ),
        out_type=jax.ShapeDtypeStruct((B, 2 * D), jnp.float32),
        scratch_types=[
            pltpu.VMEM((n_chunks, CH), jnp.int32),
            pltpu.VMEM((NBUF, CH, D), jnp.float32),
            pltpu.VMEM((2, CH, 2 * D), jnp.float32),
            pltpu.SemaphoreType.DMA,
            pltpu.SemaphoreType.DMA,
        ],
    )
    def k(idx_hbm, tab_hbm, out_hbm, idx_v, buf, outb, gsem, osem):
        wid = lax.axis_index("s") * NC + lax.axis_index("c")
        base = wid * (n_chunks * CH)
        pltpu.sync_copy(idx_hbm.at[wid], idx_v)

        def start_chunk(j, slot):
            for t in range(NSPLIT):
                pltpu.async_copy(
                    tab_hbm.at[idx_v.at[j, pl.ds(t * SUB, SUB)]],
                    buf.at[slot, pl.ds(t * SUB, SUB)],
                    gsem,
                )

        def finish_chunk(j, slot, oslot):
            for t in range(NSPLIT):
                pltpu.make_async_copy(
                    tab_hbm.at[idx_v.at[j, pl.ds(t * SUB, SUB)]],
                    buf.at[slot, pl.ds(t * SUB, SUB)],
                    gsem,
                ).wait()

            def scale_row(r, _):
                for c in range(D // 16):
                    outb[oslot, r, pl.ds(c * 16, 16)] = (
                        buf[slot, r, pl.ds(c * 16, 16)] * 8.0
                    )
                return 0

            lax.fori_loop(0, CH, scale_row, 0, unroll=4)
            pltpu.async_copy(
                outb.at[oslot],
                out_hbm.at[pl.ds(base + j * CH, CH)],
                osem,
            )

        def drain_out(j, oslot):
            pltpu.make_async_copy(
                outb.at[oslot],
                out_hbm.at[pl.ds(base + j * CH, CH)],
                osem,
            ).wait()

        for u in range(NBUF - 1):
            start_chunk(u, u)

        def body(j, _):
            for u in range(NBUF):
                jj = j * NBUF + u
                oslot = lax.rem(jj, 2)

                @pl.when(jj + NBUF - 1 < n_chunks)
                def _(jj=jj, u=u):
                    start_chunk(jj + NBUF - 1, (u + NBUF - 1) % NBUF)

                @pl.when(jj >= 2)
                def _(jj=jj, oslot=oslot):
                    drain_out(jj - 2, oslot)

                finish_chunk(jj, u, oslot)
            return 0

        lax.fori_loop(0, n_chunks // NBUF, body, 0)
        drain_out(n_chunks - 2, lax.rem(n_chunks - 2, 2))
        drain_out(n_chunks - 1, lax.rem(n_chunks - 1, 2))

    return k


def kernel(x, table):
    S0, S1 = x.shape
    V, D = table.shape
    B = S0 * S1
    n_chunks = B // (NW * CH)
    xi = x.astype(jnp.int32)
    idx = xi.reshape(NW, n_chunks, CH)
    g = _gather_kernel(B, V, D, n_chunks)(idx, table)
    return g[:, :D].reshape(S0, S1, D)
